# baseline (device time: 57669 ns/iter reference)
import jax
import jax.numpy as jnp
from jax import lax
from jax.experimental import pallas as pl
from jax.experimental.pallas import tpu as pltpu

N_DEV = 4
N_TOK = 2048
D_IN = 512
D_OUT = 1024
N_EXP = 32
N_EXP_LOCAL = 8
CAP = 51
CAP_PAD = 56
SLOTS = N_EXP_LOCAL * CAP_PAD


def kernel(x, router_W, route_idx, expert_W):
    del router_W

    def body(x_ref, idx_ref, w_ref, out_ref, ybuf, send_sems, recv_sems):
        p = lax.axis_index("i")
        left = lax.rem(p + N_DEV - 1, N_DEV)
        right = lax.rem(p + 1, N_DEV)

        diag = lax.rem(p + 2, N_DEV)

        barrier_sem = pltpu.get_barrier_semaphore()
        for nbr in (left, right, diag):
            pl.semaphore_signal(
                barrier_sem, inc=1,
                device_id=(nbr,), device_id_type=pl.DeviceIdType.MESH,
            )
        pl.semaphore_wait(barrier_sem, 3)

        tri = (
            lax.broadcasted_iota(jnp.int32, (N_TOK, N_TOK), 0)
            >= lax.broadcasted_iota(jnp.int32, (N_TOK, N_TOK), 1)
        ).astype(jnp.bfloat16)
        eids = lax.broadcasted_iota(jnp.int32, (N_TOK, N_EXP), 1)

        idx_all = idx_ref[:, :]
        oh = (idx_all == eids).astype(jnp.bfloat16)
        csum = jnp.dot(tri, oh, preferred_element_type=jnp.float32)
        sel = jnp.sum(oh * csum, axis=1, keepdims=True)
        keep = sel <= float(CAP)
        rank = sel.astype(jnp.int32) - 1
        slot = jnp.where(keep, idx_all * CAP_PAD + rank, -1)

        slot_iota = lax.broadcasted_iota(jnp.int32, (N_TOK, SLOTS), 1)

        d_mine = ((slot - p * SLOTS) == slot_iota).astype(jnp.bfloat16)
        xb = x_ref[:, :].astype(jnp.bfloat16)
        xg = lax.dot_general(
            d_mine, xb, (((0,), (0,)), ((), ())),
            preferred_element_type=jnp.float32,
        ).astype(jnp.bfloat16)
        for k in range(N_EXP_LOCAL):
            wk = w_ref[k].astype(jnp.bfloat16)
            ybuf[0, pl.ds(k * CAP_PAD, CAP_PAD), :] = jnp.dot(
                xg[k * CAP_PAD:(k + 1) * CAP_PAD, :], wk,
                preferred_element_type=jnp.float32,
            ).astype(jnp.bfloat16)

        sends = []
        for i, (tgt, slot_id) in enumerate(((left, 1), (right, 2), (diag, 3))):
            rdma = pltpu.make_async_remote_copy(
                src_ref=ybuf.at[0],
                dst_ref=ybuf.at[slot_id],
                send_sem=send_sems.at[i],
                recv_sem=recv_sems.at[slot_id - 1],
                device_id=(tgt,),
                device_id_type=pl.DeviceIdType.MESH,
            )
            rdma.start()
            sends.append(rdma)

        out_ref[:, :] = jnp.dot(
            d_mine, ybuf[0], preferred_element_type=jnp.float32
        )

        for slot_id, q in ((1, right), (2, left), (3, diag)):
            d_q = ((slot - q * SLOTS) == slot_iota).astype(jnp.bfloat16)
            recv = pltpu.make_async_remote_copy(
                src_ref=ybuf.at[0],
                dst_ref=ybuf.at[slot_id],
                send_sem=send_sems.at[0],
                recv_sem=recv_sems.at[slot_id - 1],
                device_id=(q,),
                device_id_type=pl.DeviceIdType.MESH,
            )
            recv.wait_recv()
            out_ref[:, :] = out_ref[:, :] + jnp.dot(
                d_q, ybuf[slot_id], preferred_element_type=jnp.float32
            )

        for rdma in sends:
            rdma.wait_send()

    return pl.pallas_call(
        body,
        out_shape=jax.ShapeDtypeStruct((N_TOK, D_OUT), jnp.float32),
        in_specs=[
            pl.BlockSpec(memory_space=pltpu.VMEM),
            pl.BlockSpec(memory_space=pltpu.VMEM),
            pl.BlockSpec(memory_space=pltpu.VMEM),
        ],
        out_specs=pl.BlockSpec(memory_space=pltpu.VMEM),
        scratch_shapes=[
            pltpu.VMEM((N_DEV, SLOTS, D_OUT), jnp.bfloat16),
            pltpu.SemaphoreType.DMA((N_DEV - 1,)),
            pltpu.SemaphoreType.DMA((N_DEV - 1,)),
        ],
        compiler_params=pltpu.CompilerParams(
            collective_id=0,
            vmem_limit_bytes=100 * 1024 * 1024,
        ),
    )(x, route_idx, expert_W)


# device time: 51498 ns/iter; 1.1198x vs baseline; 1.1198x over previous
import jax
import jax.numpy as jnp
from jax import lax
from jax.experimental import pallas as pl
from jax.experimental.pallas import tpu as pltpu

N_DEV = 4
N_TOK = 2048
D_IN = 512
D_OUT = 1024
N_EXP = 32
N_EXP_LOCAL = 8
CAP = 51
CAP_PAD = 56
SLOTS = N_EXP_LOCAL * CAP_PAD


def kernel(x, router_W, route_idx, expert_W):
    del router_W

    def body(x_ref, idx_ref, w_ref, out_ref, ybuf, send_sems, recv_sems):
        p = lax.axis_index("i")
        left = lax.rem(p + N_DEV - 1, N_DEV)
        right = lax.rem(p + 1, N_DEV)

        diag = lax.rem(p + 2, N_DEV)

        barrier_sem = pltpu.get_barrier_semaphore()
        for nbr in (left, right, diag):
            pl.semaphore_signal(
                barrier_sem, inc=1,
                device_id=(nbr,), device_id_type=pl.DeviceIdType.MESH,
            )
        pl.semaphore_wait(barrier_sem, 3)

        CHUNK = 512
        tri = (
            lax.broadcasted_iota(jnp.int32, (CHUNK, CHUNK), 0)
            >= lax.broadcasted_iota(jnp.int32, (CHUNK, CHUNK), 1)
        ).astype(jnp.bfloat16)
        eids = lax.broadcasted_iota(jnp.int32, (CHUNK, N_EXP), 1)

        prev = jnp.zeros((1, N_EXP), jnp.float32)
        slot_chunks = []
        for c in range(N_TOK // CHUNK):
            idx_c = idx_ref[pl.ds(c * CHUNK, CHUNK), :]
            oh = (idx_c == eids).astype(jnp.bfloat16)
            csum = jnp.dot(tri, oh, preferred_element_type=jnp.float32) + prev
            prev = prev + jnp.sum(oh.astype(jnp.float32), axis=0, keepdims=True)
            sel = jnp.sum(oh * csum, axis=1, keepdims=True)
            keep = sel <= float(CAP)
            rank = sel.astype(jnp.int32) - 1
            slot_chunks.append(
                jnp.where(keep, idx_c * CAP_PAD + rank, -1)
            )
        slot = jnp.concatenate(slot_chunks, axis=0)

        slot_iota = lax.broadcasted_iota(jnp.int32, (N_TOK, SLOTS), 1)

        d_mine = ((slot - p * SLOTS) == slot_iota).astype(jnp.bfloat16)
        xb = x_ref[:, :].astype(jnp.bfloat16)
        xg = lax.dot_general(
            d_mine, xb, (((0,), (0,)), ((), ())),
            preferred_element_type=jnp.float32,
        ).astype(jnp.bfloat16)
        for k in range(N_EXP_LOCAL):
            wk = w_ref[k].astype(jnp.bfloat16)
            ybuf[0, pl.ds(k * CAP_PAD, CAP_PAD), :] = jnp.dot(
                xg[k * CAP_PAD:(k + 1) * CAP_PAD, :], wk,
                preferred_element_type=jnp.float32,
            ).astype(jnp.bfloat16)

        sends = []
        for i, (tgt, slot_id) in enumerate(((left, 1), (right, 2), (diag, 3))):
            rdma = pltpu.make_async_remote_copy(
                src_ref=ybuf.at[0],
                dst_ref=ybuf.at[slot_id],
                send_sem=send_sems.at[i],
                recv_sem=recv_sems.at[slot_id - 1],
                device_id=(tgt,),
                device_id_type=pl.DeviceIdType.MESH,
            )
            rdma.start()
            sends.append(rdma)

        out_ref[:, :] = jnp.dot(
            d_mine, ybuf[0], preferred_element_type=jnp.float32
        )

        for slot_id, q in ((1, right), (2, left), (3, diag)):
            d_q = ((slot - q * SLOTS) == slot_iota).astype(jnp.bfloat16)
            recv = pltpu.make_async_remote_copy(
                src_ref=ybuf.at[0],
                dst_ref=ybuf.at[slot_id],
                send_sem=send_sems.at[0],
                recv_sem=recv_sems.at[slot_id - 1],
                device_id=(q,),
                device_id_type=pl.DeviceIdType.MESH,
            )
            recv.wait_recv()
            out_ref[:, :] = out_ref[:, :] + jnp.dot(
                d_q, ybuf[slot_id], preferred_element_type=jnp.float32
            )

        for rdma in sends:
            rdma.wait_send()

    return pl.pallas_call(
        body,
        out_shape=jax.ShapeDtypeStruct((N_TOK, D_OUT), jnp.float32),
        in_specs=[
            pl.BlockSpec(memory_space=pltpu.VMEM),
            pl.BlockSpec(memory_space=pltpu.VMEM),
            pl.BlockSpec(memory_space=pltpu.VMEM),
        ],
        out_specs=pl.BlockSpec(memory_space=pltpu.VMEM),
        scratch_shapes=[
            pltpu.VMEM((N_DEV, SLOTS, D_OUT), jnp.bfloat16),
            pltpu.SemaphoreType.DMA((N_DEV - 1,)),
            pltpu.SemaphoreType.DMA((N_DEV - 1,)),
        ],
        compiler_params=pltpu.CompilerParams(
            collective_id=0,
            vmem_limit_bytes=100 * 1024 * 1024,
        ),
    )(x, route_idx, expert_W)


# device time: 50230 ns/iter; 1.1481x vs baseline; 1.0252x over previous
import jax
import jax.numpy as jnp
from jax import lax
from jax.experimental import pallas as pl
from jax.experimental.pallas import tpu as pltpu

N_DEV = 4
N_TOK = 2048
D_IN = 512
D_OUT = 1024
N_EXP = 32
N_EXP_LOCAL = 8
CAP = 51
CAP_PAD = 56
SLOTS = N_EXP_LOCAL * CAP_PAD


def kernel(x, router_W, route_idx, expert_W):
    del router_W

    def body(x_ref, idx_ref, w_ref, out_ref, ybuf, send_sems, recv_sems):
        p = lax.axis_index("i")
        left = lax.rem(p + N_DEV - 1, N_DEV)
        right = lax.rem(p + 1, N_DEV)

        diag = lax.rem(p + 2, N_DEV)

        barrier_sem = pltpu.get_barrier_semaphore()
        for nbr in (left, right, diag):
            pl.semaphore_signal(
                barrier_sem, inc=1,
                device_id=(nbr,), device_id_type=pl.DeviceIdType.MESH,
            )
        pl.semaphore_wait(barrier_sem, 3)

        CHUNK = 512
        tri = (
            lax.broadcasted_iota(jnp.int32, (CHUNK, CHUNK), 0)
            >= lax.broadcasted_iota(jnp.int32, (CHUNK, CHUNK), 1)
        ).astype(jnp.bfloat16)
        eids = lax.broadcasted_iota(jnp.int32, (CHUNK, N_EXP), 1)

        prev = jnp.zeros((1, N_EXP), jnp.float32)
        slot_chunks = []
        for c in range(N_TOK // CHUNK):
            idx_c = idx_ref[pl.ds(c * CHUNK, CHUNK), :]
            oh = (idx_c == eids).astype(jnp.bfloat16)
            csum = jnp.dot(tri, oh, preferred_element_type=jnp.float32) + prev
            prev = prev + jnp.sum(oh.astype(jnp.float32), axis=0, keepdims=True)
            sel = jnp.sum(oh * csum, axis=1, keepdims=True)
            keep = sel <= float(CAP)
            rank = sel.astype(jnp.int32) - 1
            slot_chunks.append(
                jnp.where(keep, idx_c * CAP_PAD + rank, -1)
            )
        slot = jnp.concatenate(slot_chunks, axis=0)

        slot_iota = lax.broadcasted_iota(jnp.int32, (N_TOK, SLOTS), 1)

        slot_row = lax.transpose(slot, (1, 0))
        d_mine_t = (
            (slot_row - p * SLOTS)
            == lax.broadcasted_iota(jnp.int32, (SLOTS, N_TOK), 0)
        ).astype(jnp.bfloat16)
        xb = x_ref[:, :].astype(jnp.bfloat16)
        xg = jnp.dot(
            d_mine_t, xb, preferred_element_type=jnp.float32
        ).astype(jnp.bfloat16)
        HROW = SLOTS // 2
        sends = []
        for h in range(2):
            for k in range(h * 4, h * 4 + 4):
                wk = w_ref[k].astype(jnp.bfloat16)
                ybuf[0, pl.ds(k * CAP_PAD, CAP_PAD), :] = jnp.dot(
                    xg[k * CAP_PAD:(k + 1) * CAP_PAD, :], wk,
                    preferred_element_type=jnp.float32,
                ).astype(jnp.bfloat16)
            rows = pl.ds(h * HROW, HROW)
            for i, (tgt, slot_id) in enumerate(
                ((left, 1), (right, 2), (diag, 3))
            ):
                rdma = pltpu.make_async_remote_copy(
                    src_ref=ybuf.at[0, rows],
                    dst_ref=ybuf.at[slot_id, rows],
                    send_sem=send_sems.at[2 * i + h],
                    recv_sem=recv_sems.at[2 * (slot_id - 1) + h],
                    device_id=(tgt,),
                    device_id_type=pl.DeviceIdType.MESH,
                )
                rdma.start()
                sends.append(rdma)

        def wait_slot(slot_id, q):
            for h in range(2):
                rows = pl.ds(h * HROW, HROW)
                recv = pltpu.make_async_remote_copy(
                    src_ref=ybuf.at[0, rows],
                    dst_ref=ybuf.at[slot_id, rows],
                    send_sem=send_sems.at[0],
                    recv_sem=recv_sems.at[2 * (slot_id - 1) + h],
                    device_id=(q,),
                    device_id_type=pl.DeviceIdType.MESH,
                )
                recv.wait_recv()

        d_mine = ((slot - p * SLOTS) == slot_iota).astype(jnp.bfloat16)
        out_ref[:, :] = jnp.dot(
            d_mine, ybuf[0], preferred_element_type=jnp.float32
        )

        col_rl = jnp.where(
            jax.lax.div(slot, SLOTS) == right,
            slot - right * SLOTS,
            jnp.where(
                jax.lax.div(slot, SLOTS) == left,
                SLOTS + (slot - left * SLOTS),
                -1,
            ),
        )
        d_rl = (
            col_rl == lax.broadcasted_iota(jnp.int32, (N_TOK, 2 * SLOTS), 1)
        ).astype(jnp.bfloat16)
        d_diag = ((slot - diag * SLOTS) == slot_iota).astype(jnp.bfloat16)

        wait_slot(1, right)
        wait_slot(2, left)
        y_rl = ybuf[pl.ds(1, 2)].reshape(2 * SLOTS, D_OUT)
        out_ref[:, :] = out_ref[:, :] + jnp.dot(
            d_rl, y_rl, preferred_element_type=jnp.float32
        )

        wait_slot(3, diag)
        out_ref[:, :] = out_ref[:, :] + jnp.dot(
            d_diag, ybuf[3], preferred_element_type=jnp.float32
        )

        for rdma in sends:
            rdma.wait_send()

    return pl.pallas_call(
        body,
        out_shape=jax.ShapeDtypeStruct((N_TOK, D_OUT), jnp.float32),
        in_specs=[
            pl.BlockSpec(memory_space=pltpu.VMEM),
            pl.BlockSpec(memory_space=pltpu.VMEM),
            pl.BlockSpec(memory_space=pltpu.VMEM),
        ],
        out_specs=pl.BlockSpec(memory_space=pltpu.VMEM),
        scratch_shapes=[
            pltpu.VMEM((N_DEV, SLOTS, D_OUT), jnp.bfloat16),
            pltpu.SemaphoreType.DMA((6,)),
            pltpu.SemaphoreType.DMA((6,)),
        ],
        compiler_params=pltpu.CompilerParams(
            collective_id=0,
            vmem_limit_bytes=100 * 1024 * 1024,
        ),
    )(x, route_idx, expert_W)
